# baseline (speedup 1.0000x reference)
import functools

import jax
import jax.numpy as jnp
from jax import lax
from jax.experimental import pallas as pl
from jax.experimental.pallas import tpu as pltpu
from jax.experimental.pallas import tpu_sc as plsc

_N = 16384
_C = 1000
_D = 128

_COLS_PER_BLOCK = 2048

_NW = 32
_BPW = _N // _NW
_CHUNK = 128
_NCHUNK = _BPW // _CHUNK


def _argmax_block(xt_ref, idx_ref):
    x = xt_ref[...]
    maxv = jnp.max(x, axis=0, keepdims=True)
    cls = lax.broadcasted_iota(jnp.int32, x.shape, 0)
    cand = jnp.where(x == maxv, cls, jnp.int32(_C))
    idx_ref[...] = jnp.min(cand, axis=0)


@functools.cache
def _make_sc_gather():
    mesh = plsc.VectorSubcoreMesh(core_axis_name="c", subcore_axis_name="s")

    @pl.kernel(
        mesh=mesh,
        out_type=jax.ShapeDtypeStruct((_N, _D), jnp.float32),
        scratch_types=[
            pltpu.VMEM((_NCHUNK, _CHUNK), jnp.int32),
            pltpu.VMEM((_BPW, _D), jnp.float32),
            pltpu.SemaphoreType.DMA((_NCHUNK,)),
            pltpu.SemaphoreType.DMA((_NCHUNK,)),
        ],
    )
    def gather(idx_hbm, table_hbm, out_hbm, idx_v, rows_v, gsems, osems):
        w = lax.axis_index("s") * 2 + lax.axis_index("c")
        base = w * _BPW
        pltpu.sync_copy(idx_hbm.at[w], idx_v)
        copies = [
            pltpu.async_copy(
                table_hbm.at[idx_v.at[j]],
                rows_v.at[pl.ds(j * _CHUNK, _CHUNK)],
                gsems.at[j],
            )
            for j in range(_NCHUNK)
        ]
        outs = []
        for j, cp in enumerate(copies):
            cp.wait()
            outs.append(pltpu.async_copy(
                rows_v.at[pl.ds(j * _CHUNK, _CHUNK)],
                out_hbm.at[pl.ds(base + j * _CHUNK, _CHUNK)],
                osems.at[j],
            ))
        for cp in outs:
            cp.wait()

    return gather


def kernel(frag_attr, embedding_weight):
    xt = frag_attr.T
    idx = pl.pallas_call(
        _argmax_block,
        grid=(_N // _COLS_PER_BLOCK,),
        in_specs=[pl.BlockSpec((_C, _COLS_PER_BLOCK), lambda i: (0, i))],
        out_specs=pl.BlockSpec((_COLS_PER_BLOCK,), lambda i: (i,)),
        out_shape=jax.ShapeDtypeStruct((_N,), jnp.int32),
    )(xt)
    idx3 = idx.reshape(_NW, _NCHUNK, _CHUNK)
    return _make_sc_gather()(idx3, embedding_weight)


# speedup vs baseline: 1.0436x; 1.0436x over previous
import functools

import jax
import jax.numpy as jnp
from jax import lax
from jax.experimental import pallas as pl
from jax.experimental.pallas import tpu as pltpu
from jax.experimental.pallas import tpu_sc as plsc

_N = 16384
_C = 1000
_D = 128

_COLS_PER_BLOCK = 2048

_NW = 32
_BPW = _N // _NW
_CHUNK = 128
_NCHUNK = _BPW // _CHUNK


def _argmax_block(xt_ref, idx_ref):
    x = xt_ref[...]
    maxv = jnp.max(x, axis=0, keepdims=True)
    cls = lax.broadcasted_iota(jnp.int32, x.shape, 0)
    cand = jnp.where(x == maxv, cls, jnp.int32(_C))
    idx_ref[...] = jnp.min(cand, axis=0)


@functools.cache
def _make_sc_gather():
    mesh = plsc.VectorSubcoreMesh(core_axis_name="c", subcore_axis_name="s")

    @pl.kernel(
        mesh=mesh,
        out_type=jax.ShapeDtypeStruct((_N, _D), jnp.float32),
        scratch_types=[
            pltpu.VMEM((_NCHUNK, _CHUNK), jnp.int32),
            pltpu.VMEM((_BPW, _D), jnp.float32),
            pltpu.SemaphoreType.DMA,
        ],
    )
    def gather(idx_hbm, table_hbm, out_hbm, idx_v, rows_v, sem):
        w = lax.axis_index("s") * 2 + lax.axis_index("c")
        base = w * _BPW
        pltpu.sync_copy(idx_hbm.at[w], idx_v)
        copies = [
            pltpu.async_copy(
                table_hbm.at[idx_v.at[j]],
                rows_v.at[pl.ds(j * _CHUNK, _CHUNK)],
                sem,
            )
            for j in range(_NCHUNK)
        ]
        for cp in copies:
            cp.wait()
        pltpu.sync_copy(rows_v, out_hbm.at[pl.ds(base, _BPW)])

    return gather


def kernel(frag_attr, embedding_weight):
    xt = frag_attr.T
    idx = pl.pallas_call(
        _argmax_block,
        grid=(_N // _COLS_PER_BLOCK,),
        in_specs=[pl.BlockSpec((_C, _COLS_PER_BLOCK), lambda i: (0, i))],
        out_specs=pl.BlockSpec((_COLS_PER_BLOCK,), lambda i: (i,)),
        out_shape=jax.ShapeDtypeStruct((_N,), jnp.int32),
    )(xt)
    idx3 = idx.reshape(_NW, _NCHUNK, _CHUNK)
    return _make_sc_gather()(idx3, embedding_weight)
